# baseline (speedup 1.0000x reference)
import functools

import jax
import jax.numpy as jnp
from jax import lax
from jax.experimental import pallas as pl
from jax.experimental.pallas import tpu as pltpu
from jax.experimental.pallas import tpu_sc as plsc

N = 10000
NP = 10240
D = 128
G = 128
E = 320000
ETOT = E + N
NC, NS, L = 2, 16, 16
DH = D // NC
K = 128
EPW = ((ETOT + 3 * NS * K - 1) // (3 * NS * K)) * 3 * K
EPAD = EPW * NS
NB = EPW // K
ROWS_PER_TILE = NP // NS
DUMMY = N

_f32 = jnp.float32


def _mm_attn_body(x_ref, w_ref, as_ref, ad_ref, h_ref, s_ref, d_ref):
    h = jnp.dot(x_ref[...], w_ref[...], preferred_element_type=_f32)
    h_ref[0] = h[:, :DH]
    h_ref[1] = h[:, DH:]
    s_ref[...] = jnp.dot(h, as_ref[...], preferred_element_type=_f32)
    d_ref[...] = jnp.dot(h, ad_ref[...], preferred_element_type=_f32)


def _finalize(num_ref, den_ref, b_ref):
    nsum = jnp.concatenate([num_ref[0], num_ref[1]], axis=1)
    ones_col = jnp.ones((NS, 1), _f32)
    dcol = lax.dot_general(den_ref[...], ones_col,
                           (((0,), (0,)), ((), ())),
                           preferred_element_type=_f32)
    return jax.nn.relu(nsum / (dcol + 1e-16) + b_ref[...])


def _fin_mm_attn_body(num_ref, den_ref, b_ref, w_ref, as_ref, ad_ref,
                      h_ref, s_ref, d_ref):
    xin = _finalize(num_ref, den_ref, b_ref)
    h = jnp.dot(xin, w_ref[...], preferred_element_type=_f32)
    h_ref[0] = h[:, :DH]
    h_ref[1] = h[:, DH:]
    s_ref[...] = jnp.dot(h, as_ref[...], preferred_element_type=_f32)
    d_ref[...] = jnp.dot(h, ad_ref[...], preferred_element_type=_f32)


def _pool_body(num_ref, den_ref, b_ref, batch_ref, out_ref, sums_sc, cnt_sc):
    i = pl.program_id(0)
    h2 = _finalize(num_ref, den_ref, b_ref)
    gids = lax.broadcasted_iota(jnp.int32, (1024, G), 1)
    mf = (batch_ref[...] == gids).astype(_f32)
    psum = lax.dot_general(mf, h2, (((0,), (0,)), ((), ())),
                           preferred_element_type=_f32)
    pcnt = lax.dot_general(mf, jnp.ones((1024, D), _f32),
                           (((0,), (0,)), ((), ())),
                           preferred_element_type=_f32)

    @pl.when(i == 0)
    def _init():
        sums_sc[...] = jnp.zeros((G, D), _f32)
        cnt_sc[...] = jnp.zeros((G, D), _f32)

    sums_sc[...] += psum
    cnt_sc[...] += pcnt

    @pl.when(i == NP // 1024 - 1)
    def _done():
        out_ref[...] = sums_sc[...] / jnp.maximum(cnt_sc[...], 1.0)


_ROW = pl.BlockSpec((1024, D), lambda i: (i, 0))
_FULL_W = pl.BlockSpec((D, D), lambda i: (0, 0))
_COL = pl.BlockSpec((D, 1), lambda i: (0, 0))
_SCOL = pl.BlockSpec((1024, 1), lambda i: (i, 0))
_NUMS = pl.BlockSpec((NC, 1024, DH), lambda i: (0, i, 0))
_DENS = pl.BlockSpec((NS, 1024), lambda i: (0, i))
_HOUT = pl.BlockSpec((NC, 1024, DH), lambda i: (0, i, 0))
_BROW = pl.BlockSpec((1, D), lambda i: (0, 0))

_GRID = (NP // 1024,)

_mm_attn = pl.pallas_call(
    _mm_attn_body,
    grid=_GRID,
    in_specs=[_ROW, _FULL_W, _COL, _COL],
    out_specs=[_HOUT, _SCOL, _SCOL],
    out_shape=[jax.ShapeDtypeStruct((NC, NP, DH), _f32),
               jax.ShapeDtypeStruct((NP, 1), _f32),
               jax.ShapeDtypeStruct((NP, 1), _f32)],
)

_fin_mm_attn = pl.pallas_call(
    _fin_mm_attn_body,
    grid=_GRID,
    in_specs=[_NUMS, _DENS, _BROW, _FULL_W, _COL, _COL],
    out_specs=[_HOUT, _SCOL, _SCOL],
    out_shape=[jax.ShapeDtypeStruct((NC, NP, DH), _f32),
               jax.ShapeDtypeStruct((NP, 1), _f32),
               jax.ShapeDtypeStruct((NP, 1), _f32)],
)

_pool = pl.pallas_call(
    _pool_body,
    grid=_GRID,
    in_specs=[_NUMS, _DENS, _BROW, _SCOL],
    out_specs=pl.BlockSpec((G, D), lambda i: (0, 0)),
    out_shape=jax.ShapeDtypeStruct((G, D), _f32),
    scratch_shapes=[pltpu.VMEM((G, D), _f32), pltpu.VMEM((G, D), _f32)],
)


@functools.lru_cache(maxsize=1)
def _build_edge_phase():
  mesh = plsc.VectorSubcoreMesh(core_axis_name="c", subcore_axis_name="s",
                                num_cores=NC, num_subcores=NS)

  @functools.partial(
    pl.kernel,
    out_type=[jax.ShapeDtypeStruct((NC, NP, DH), _f32),
              jax.ShapeDtypeStruct((NS, NP), _f32)],
    mesh=mesh,
    scratch_types=[
        pltpu.VMEM((NP,), _f32),
        pltpu.VMEM((NP,), _f32),
        pltpu.VMEM((NP,), _f32),
        [pltpu.VMEM((K,), jnp.int32)] * 3,
        [pltpu.VMEM((K,), jnp.int32)] * 3,
        pltpu.VMEM((K,), _f32),
        [pltpu.VMEM((K, DH), _f32)] * 3,
        pltpu.VMEM_SHARED((NP, DH), _f32),
        [pltpu.SemaphoreType.DMA] * 3,
        [pltpu.SemaphoreType.DMA] * 3,
    ],
    compiler_params=pltpu.CompilerParams(needs_layout_passes=False,
                                         use_tc_tiling_on_sc=False),
  )
  def _edge_phase(h_hbm, s_hbm, d_hbm, comb_hbm, num_out, den_out,
                  s_v, d_v, den_v, src_v, dst_v, p_v, rows_v, acc, gsem, ssem):
      cc = lax.axis_index("c")
      ss = lax.axis_index("s")

      pltpu.sync_copy(s_hbm, s_v)
      pltpu.sync_copy(d_hbm, d_v)

      zero16 = jnp.zeros((L,), _f32)

      def _zden(i, _):
          den_v[pl.ds(i * L, L)] = zero16
          return 0
      lax.fori_loop(0, NP // L, _zden, 0)

      def _zrows(i, _):
          for r in range(DH // L):
              rows_v[0][i, pl.ds(r * L, L)] = zero16
          return 0
      lax.fori_loop(0, K, _zrows, 0)
      for j in range(ROWS_PER_TILE // K):
          pltpu.sync_copy(rows_v[0],
                          acc.at[pl.ds(ss * ROWS_PER_TILE + j * K, K)])
      plsc.subcore_barrier()

      def _fire(bi, slot):
          base = ss * EPW + bi * K
          pltpu.sync_copy(comb_hbm.at[pl.ds(base, K)], src_v[slot])
          for j in range(K // L):
              c = src_v[slot][pl.ds(j * L, L)]
              dst_v[slot][pl.ds(j * L, L)] = lax.shift_right_logical(c, 14)
              src_v[slot][pl.ds(j * L, L)] = lax.bitwise_and(c, 16383)
          pltpu.async_copy(h_hbm.at[cc].at[src_v[slot]], rows_v[slot],
                           gsem[slot])

      def _wait_gather(slot):
          pltpu.make_async_copy(h_hbm.at[cc].at[src_v[slot]], rows_v[slot],
                                gsem[slot]).wait()

      def _wait_scatter(slot):
          pltpu.make_async_copy(rows_v[slot], acc.at[dst_v[slot]],
                                ssem[slot]).wait()

      def _process(slot):
          for j in range(K // L):
              si = src_v[slot][pl.ds(j * L, L)]
              di = dst_v[slot][pl.ds(j * L, L)]
              e = plsc.load_gather(s_v, [si]) + plsc.load_gather(d_v, [di])
              e = jnp.where(e >= 0.0, e, 0.2 * e)
              p = jnp.exp(e)
              p_v[pl.ds(j * L, L)] = p
              plsc.addupdate_scatter(den_v, [di], p)

          def _scale(g, _):
              pvec = p_v[pl.ds(g * L, L)]
              for l in range(L):
                  pv = pvec[l]
                  ei = g * L + l
                  for r in range(DH // L):
                      rows_v[slot][ei, pl.ds(r * L, L)] = (
                          rows_v[slot][ei, pl.ds(r * L, L)] * pv)
              return 0
          lax.fori_loop(0, K // L, _scale, 0)

          pltpu.async_copy(rows_v[slot], acc.at[dst_v[slot]], ssem[slot],
                           add=True)

      _fire(0, 0)
      _fire(1, 1)

      def _tri(g, _):
          for t in range(3):
              i = 3 * g + t
              _wait_gather(t)
              _process(t)
              nxt = (t + 2) % 3
              if t == 0:
                  @pl.when(g > 0)
                  def _w():
                      _wait_scatter(nxt)
              else:
                  _wait_scatter(nxt)

              @pl.when(i + 2 < NB)
              def _f():
                  _fire(i + 2, nxt)
          return 0
      lax.fori_loop(0, NB // 3, _tri, 0)
      _wait_scatter(2)

      plsc.subcore_barrier()

      @pl.when(cc == 0)
      def _wden():
          pltpu.sync_copy(den_v, den_out.at[ss])
      for j in range(ROWS_PER_TILE // K):
          off = ss * ROWS_PER_TILE + j * K
          pltpu.sync_copy(acc.at[pl.ds(off, K)], num_out.at[cc, pl.ds(off, K)])

  return _edge_phase


def kernel(x, edge_index, batch, W1, a_src1, a_dst1, b1, W2, a_src2, a_dst2, b2):
    loop = jnp.arange(N, dtype=jnp.int32)
    pad = EPAD - ETOT
    src = jnp.concatenate([edge_index[0], loop,
                           jnp.zeros((pad,), jnp.int32)])
    dst = jnp.concatenate([edge_index[1], loop,
                           jnp.full((pad,), DUMMY, jnp.int32)])
    comb = src | (dst << 14)

    x_pad = jnp.pad(x, ((0, NP - N), (0, 0)))
    batch_col = jnp.pad(batch, (0, NP - N), constant_values=G).reshape(NP, 1)
    as1 = a_src1.reshape(D, 1)
    ad1 = a_dst1.reshape(D, 1)
    as2 = a_src2.reshape(D, 1)
    ad2 = a_dst2.reshape(D, 1)
    b1r = b1.reshape(1, D)
    b2r = b2.reshape(1, D)

    edge_phase = _build_edge_phase()
    h1, s1, d1 = _mm_attn(x_pad, W1, as1, ad1)
    num1, den1 = edge_phase(h1, s1.reshape(NP), d1.reshape(NP), comb)
    h2, s2, d2 = _fin_mm_attn(num1, den1, b1r, W2, as2, ad2)
    num2, den2 = edge_phase(h2, s2.reshape(NP), d2.reshape(NP), comb)
    return _pool(num2, den2, b2r, batch_col)


# speedup vs baseline: 1.1263x; 1.1263x over previous
import functools

import jax
import jax.numpy as jnp
from jax import lax
from jax.experimental import pallas as pl
from jax.experimental.pallas import tpu as pltpu
from jax.experimental.pallas import tpu_sc as plsc

N = 10000
NP = 10240
D = 128
G = 128
E = 320000
ETOT = E + N
NC, NS, L = 2, 16, 16
DH = D // NC
K = 128
EPW = ((ETOT + 3 * NS * K - 1) // (3 * NS * K)) * 3 * K
EPAD = EPW * NS
NB = EPW // K
ROWS_PER_TILE = NP // NS
DUMMY = N

_f32 = jnp.float32


def _mm_attn_body(x_ref, w_ref, as_ref, ad_ref, h_ref, s_ref, d_ref):
    h = jnp.dot(x_ref[...], w_ref[...], preferred_element_type=_f32)
    h_ref[0] = h[:, :DH]
    h_ref[1] = h[:, DH:]
    s_ref[...] = jnp.dot(h, as_ref[...], preferred_element_type=_f32)
    d_ref[...] = jnp.dot(h, ad_ref[...], preferred_element_type=_f32)


def _finalize(num_ref, den_ref, b_ref):
    nsum = jnp.concatenate([num_ref[0], num_ref[1]], axis=1)
    ones_col = jnp.ones((NS, 1), _f32)
    dcol = lax.dot_general(den_ref[...], ones_col,
                           (((0,), (0,)), ((), ())),
                           preferred_element_type=_f32)
    return jax.nn.relu(nsum / (dcol + 1e-16) + b_ref[...])


def _fin_mm_attn_body(num_ref, den_ref, b_ref, w_ref, as_ref, ad_ref,
                      h_ref, s_ref, d_ref):
    xin = _finalize(num_ref, den_ref, b_ref)
    h = jnp.dot(xin, w_ref[...], preferred_element_type=_f32)
    h_ref[0] = h[:, :DH]
    h_ref[1] = h[:, DH:]
    s_ref[...] = jnp.dot(h, as_ref[...], preferred_element_type=_f32)
    d_ref[...] = jnp.dot(h, ad_ref[...], preferred_element_type=_f32)


def _pool_body(num_ref, den_ref, b_ref, batch_ref, out_ref, sums_sc, cnt_sc):
    i = pl.program_id(0)
    h2 = _finalize(num_ref, den_ref, b_ref)
    gids = lax.broadcasted_iota(jnp.int32, (1024, G), 1)
    mf = (batch_ref[...] == gids).astype(_f32)
    psum = lax.dot_general(mf, h2, (((0,), (0,)), ((), ())),
                           preferred_element_type=_f32)
    pcnt = lax.dot_general(mf, jnp.ones((1024, D), _f32),
                           (((0,), (0,)), ((), ())),
                           preferred_element_type=_f32)

    @pl.when(i == 0)
    def _init():
        sums_sc[...] = jnp.zeros((G, D), _f32)
        cnt_sc[...] = jnp.zeros((G, D), _f32)

    sums_sc[...] += psum
    cnt_sc[...] += pcnt

    @pl.when(i == NP // 1024 - 1)
    def _done():
        out_ref[...] = sums_sc[...] / jnp.maximum(cnt_sc[...], 1.0)


_ROW = pl.BlockSpec((1024, D), lambda i: (i, 0))
_FULL_W = pl.BlockSpec((D, D), lambda i: (0, 0))
_COL = pl.BlockSpec((D, 1), lambda i: (0, 0))
_SCOL = pl.BlockSpec((1024, 1), lambda i: (i, 0))
_NUMS = pl.BlockSpec((NC, 1024, DH), lambda i: (0, i, 0))
_DENS = pl.BlockSpec((NS, 1024), lambda i: (0, i))
_HOUT = pl.BlockSpec((NC, 1024, DH), lambda i: (0, i, 0))
_BROW = pl.BlockSpec((1, D), lambda i: (0, 0))

_GRID = (NP // 1024,)

_mm_attn = pl.pallas_call(
    _mm_attn_body,
    grid=_GRID,
    in_specs=[_ROW, _FULL_W, _COL, _COL],
    out_specs=[_HOUT, _SCOL, _SCOL],
    out_shape=[jax.ShapeDtypeStruct((NC, NP, DH), _f32),
               jax.ShapeDtypeStruct((NP, 1), _f32),
               jax.ShapeDtypeStruct((NP, 1), _f32)],
)

_fin_mm_attn = pl.pallas_call(
    _fin_mm_attn_body,
    grid=_GRID,
    in_specs=[_NUMS, _DENS, _BROW, _FULL_W, _COL, _COL],
    out_specs=[_HOUT, _SCOL, _SCOL],
    out_shape=[jax.ShapeDtypeStruct((NC, NP, DH), _f32),
               jax.ShapeDtypeStruct((NP, 1), _f32),
               jax.ShapeDtypeStruct((NP, 1), _f32)],
)

_pool = pl.pallas_call(
    _pool_body,
    grid=_GRID,
    in_specs=[_NUMS, _DENS, _BROW, _SCOL],
    out_specs=pl.BlockSpec((G, D), lambda i: (0, 0)),
    out_shape=jax.ShapeDtypeStruct((G, D), _f32),
    scratch_shapes=[pltpu.VMEM((G, D), _f32), pltpu.VMEM((G, D), _f32)],
)


@functools.lru_cache(maxsize=1)
def _build_edge_phase():
  mesh = plsc.VectorSubcoreMesh(core_axis_name="c", subcore_axis_name="s",
                                num_cores=NC, num_subcores=NS)

  @functools.partial(
    pl.kernel,
    out_type=[jax.ShapeDtypeStruct((NC, NP, DH), _f32),
              jax.ShapeDtypeStruct((NS, NP), _f32)],
    mesh=mesh,
    scratch_types=[
        pltpu.VMEM((NP,), _f32),
        pltpu.VMEM((NP,), _f32),
        pltpu.VMEM((NP,), _f32),
        pltpu.VMEM((EPW,), jnp.int32),
        [pltpu.VMEM((K,), jnp.int32)] * 3,
        [pltpu.VMEM((K,), jnp.int32)] * 3,
        pltpu.VMEM((K,), _f32),
        [pltpu.VMEM((K, DH), _f32)] * 3,
        pltpu.VMEM_SHARED((NP, DH), _f32),
        [pltpu.SemaphoreType.DMA] * 3,
        [pltpu.SemaphoreType.DMA] * 3,
    ],
    compiler_params=pltpu.CompilerParams(needs_layout_passes=False,
                                         use_tc_tiling_on_sc=False),
  )
  def _edge_phase(h_hbm, s_hbm, d_hbm, comb_hbm, num_out, den_out,
                  s_v, d_v, den_v, comb_v, src_v, dst_v, p_v, rows_v, acc,
                  gsem, ssem):
      cc = lax.axis_index("c")
      ss = lax.axis_index("s")

      pltpu.sync_copy(s_hbm, s_v)
      pltpu.sync_copy(d_hbm, d_v)
      pltpu.sync_copy(comb_hbm.at[pl.ds(ss * EPW, EPW)], comb_v)

      zero16 = jnp.zeros((L,), _f32)

      def _zden(i, _):
          den_v[pl.ds(i * L, L)] = zero16
          return 0
      lax.fori_loop(0, NP // L, _zden, 0)

      def _zrows(i, _):
          for r in range(DH // L):
              rows_v[0][i, pl.ds(r * L, L)] = zero16
          return 0
      lax.fori_loop(0, K, _zrows, 0)
      for j in range(ROWS_PER_TILE // K):
          pltpu.sync_copy(rows_v[0],
                          acc.at[pl.ds(ss * ROWS_PER_TILE + j * K, K)])
      plsc.subcore_barrier()

      def _fire(bi, slot):
          base = bi * K
          for j in range(K // L):
              c = comb_v[pl.ds(base + j * L, L)]
              dst_v[slot][pl.ds(j * L, L)] = lax.shift_right_logical(c, 14)
              src_v[slot][pl.ds(j * L, L)] = lax.bitwise_and(c, 16383)
          pltpu.async_copy(h_hbm.at[cc].at[src_v[slot]], rows_v[slot],
                           gsem[slot])

      def _wait_gather(slot):
          pltpu.make_async_copy(h_hbm.at[cc].at[src_v[slot]], rows_v[slot],
                                gsem[slot]).wait()

      def _wait_scatter(slot):
          pltpu.make_async_copy(rows_v[slot], acc.at[dst_v[slot]],
                                ssem[slot]).wait()

      def _process(slot):
          for j in range(K // L):
              si = src_v[slot][pl.ds(j * L, L)]
              di = dst_v[slot][pl.ds(j * L, L)]
              e = plsc.load_gather(s_v, [si]) + plsc.load_gather(d_v, [di])
              e = jnp.where(e >= 0.0, e, 0.2 * e)
              p = jnp.exp(e)
              p_v[pl.ds(j * L, L)] = p
              plsc.addupdate_scatter(den_v, [di], p)

          def _scale(g, _):
              pvec = p_v[pl.ds(g * L, L)]
              for l in range(L):
                  pv = pvec[l]
                  ei = g * L + l
                  for r in range(DH // L):
                      rows_v[slot][ei, pl.ds(r * L, L)] = (
                          rows_v[slot][ei, pl.ds(r * L, L)] * pv)
              return 0
          lax.fori_loop(0, K // L, _scale, 0)

          pltpu.async_copy(rows_v[slot], acc.at[dst_v[slot]], ssem[slot],
                           add=True)

      _fire(0, 0)
      _fire(1, 1)

      def _tri(g, _):
          for t in range(3):
              i = 3 * g + t
              _wait_gather(t)
              _process(t)
              nxt = (t + 2) % 3
              if t == 0:
                  @pl.when(g > 0)
                  def _w():
                      _wait_scatter(nxt)
              else:
                  _wait_scatter(nxt)

              @pl.when(i + 2 < NB)
              def _f():
                  _fire(i + 2, nxt)
          return 0
      lax.fori_loop(0, NB // 3, _tri, 0)
      _wait_scatter(2)

      plsc.subcore_barrier()

      @pl.when(cc == 0)
      def _wden():
          pltpu.sync_copy(den_v, den_out.at[ss])
      for j in range(ROWS_PER_TILE // K):
          off = ss * ROWS_PER_TILE + j * K
          pltpu.sync_copy(acc.at[pl.ds(off, K)], num_out.at[cc, pl.ds(off, K)])

  return _edge_phase


def kernel(x, edge_index, batch, W1, a_src1, a_dst1, b1, W2, a_src2, a_dst2, b2):
    loop = jnp.arange(N, dtype=jnp.int32)
    pad = EPAD - ETOT
    src = jnp.concatenate([edge_index[0], loop,
                           jnp.zeros((pad,), jnp.int32)])
    dst = jnp.concatenate([edge_index[1], loop,
                           jnp.full((pad,), DUMMY, jnp.int32)])
    comb = src | (dst << 14)

    x_pad = jnp.pad(x, ((0, NP - N), (0, 0)))
    batch_col = jnp.pad(batch, (0, NP - N), constant_values=G).reshape(NP, 1)
    as1 = a_src1.reshape(D, 1)
    ad1 = a_dst1.reshape(D, 1)
    as2 = a_src2.reshape(D, 1)
    ad2 = a_dst2.reshape(D, 1)
    b1r = b1.reshape(1, D)
    b2r = b2.reshape(1, D)

    edge_phase = _build_edge_phase()
    h1, s1, d1 = _mm_attn(x_pad, W1, as1, ad1)
    num1, den1 = edge_phase(h1, s1.reshape(NP), d1.reshape(NP), comb)
    h2, s2, d2 = _fin_mm_attn(num1, den1, b1r, W2, as2, ad2)
    num2, den2 = edge_phase(h2, s2.reshape(NP), d2.reshape(NP), comb)
    return _pool(num2, den2, b2r, batch_col)


# speedup vs baseline: 1.8299x; 1.6247x over previous
import functools

import jax
import jax.numpy as jnp
from jax import lax
from jax.experimental import pallas as pl
from jax.experimental.pallas import tpu as pltpu
from jax.experimental.pallas import tpu_sc as plsc

N = 10000
NP = 10240
D = 128
G = 128
E = 320000
ETOT = E + N
NC, NS, L = 2, 16, 16
DH = D // NC
K = 128
EPW = ((ETOT + 3 * NS * K - 1) // (3 * NS * K)) * 3 * K
EPAD = EPW * NS
NB = EPW // K
ROWS_PER_TILE = NP // NS
DUMMY = N

_f32 = jnp.float32


def _mm_attn_body(x_ref, w_ref, as_ref, ad_ref, h_ref, s_ref, d_ref):
    h = jnp.dot(x_ref[...], w_ref[...], preferred_element_type=_f32)
    h_ref[0] = h[:, :DH]
    h_ref[1] = h[:, DH:]
    s_ref[...] = jnp.dot(h, as_ref[...], preferred_element_type=_f32)
    d_ref[...] = jnp.dot(h, ad_ref[...], preferred_element_type=_f32)


def _finalize(num_ref, den_ref, b_ref):
    nsum = jnp.concatenate([num_ref[0], num_ref[1]], axis=1)
    ones_col = jnp.ones((NS, 1), _f32)
    dcol = lax.dot_general(den_ref[...], ones_col,
                           (((0,), (0,)), ((), ())),
                           preferred_element_type=_f32)
    return jax.nn.relu(nsum / (dcol + 1e-16) + b_ref[...])


def _fin_mm_attn_body(num_ref, den_ref, b_ref, w_ref, as_ref, ad_ref,
                      h_ref, s_ref, d_ref):
    xin = _finalize(num_ref, den_ref, b_ref)
    h = jnp.dot(xin, w_ref[...], preferred_element_type=_f32)
    h_ref[0] = h[:, :DH]
    h_ref[1] = h[:, DH:]
    s_ref[...] = jnp.dot(h, as_ref[...], preferred_element_type=_f32)
    d_ref[...] = jnp.dot(h, ad_ref[...], preferred_element_type=_f32)


def _pool_body(num_ref, den_ref, b_ref, batch_ref, out_ref, sums_sc, cnt_sc):
    i = pl.program_id(0)
    h2 = _finalize(num_ref, den_ref, b_ref)
    gids = lax.broadcasted_iota(jnp.int32, (1024, G), 1)
    mf = (batch_ref[...] == gids).astype(_f32)
    psum = lax.dot_general(mf, h2, (((0,), (0,)), ((), ())),
                           preferred_element_type=_f32)
    pcnt = lax.dot_general(mf, jnp.ones((1024, D), _f32),
                           (((0,), (0,)), ((), ())),
                           preferred_element_type=_f32)

    @pl.when(i == 0)
    def _init():
        sums_sc[...] = jnp.zeros((G, D), _f32)
        cnt_sc[...] = jnp.zeros((G, D), _f32)

    sums_sc[...] += psum
    cnt_sc[...] += pcnt

    @pl.when(i == NP // 1024 - 1)
    def _done():
        out_ref[...] = sums_sc[...] / jnp.maximum(cnt_sc[...], 1.0)


_ROW = pl.BlockSpec((1024, D), lambda i: (i, 0))
_FULL_W = pl.BlockSpec((D, D), lambda i: (0, 0))
_COL = pl.BlockSpec((D, 1), lambda i: (0, 0))
_SCOL = pl.BlockSpec((1024, 1), lambda i: (i, 0))
_NUMS = pl.BlockSpec((NC, 1024, DH), lambda i: (0, i, 0))
_DENS = pl.BlockSpec((NS, 1024), lambda i: (0, i))
_HOUT = pl.BlockSpec((NC, 1024, DH), lambda i: (0, i, 0))
_BROW = pl.BlockSpec((1, D), lambda i: (0, 0))

_GRID = (NP // 1024,)

_mm_attn = pl.pallas_call(
    _mm_attn_body,
    grid=_GRID,
    in_specs=[_ROW, _FULL_W, _COL, _COL],
    out_specs=[_HOUT, _SCOL, _SCOL],
    out_shape=[jax.ShapeDtypeStruct((NC, NP, DH), _f32),
               jax.ShapeDtypeStruct((NP, 1), _f32),
               jax.ShapeDtypeStruct((NP, 1), _f32)],
)

_fin_mm_attn = pl.pallas_call(
    _fin_mm_attn_body,
    grid=_GRID,
    in_specs=[_NUMS, _DENS, _BROW, _FULL_W, _COL, _COL],
    out_specs=[_HOUT, _SCOL, _SCOL],
    out_shape=[jax.ShapeDtypeStruct((NC, NP, DH), _f32),
               jax.ShapeDtypeStruct((NP, 1), _f32),
               jax.ShapeDtypeStruct((NP, 1), _f32)],
)

_pool = pl.pallas_call(
    _pool_body,
    grid=_GRID,
    in_specs=[_NUMS, _DENS, _BROW, _SCOL],
    out_specs=pl.BlockSpec((G, D), lambda i: (0, 0)),
    out_shape=jax.ShapeDtypeStruct((G, D), _f32),
    scratch_shapes=[pltpu.VMEM((G, D), _f32), pltpu.VMEM((G, D), _f32)],
)


@functools.lru_cache(maxsize=1)
def _build_edge_phase():
  mesh = plsc.VectorSubcoreMesh(core_axis_name="c", subcore_axis_name="s",
                                num_cores=NC, num_subcores=NS)

  @functools.partial(
    pl.kernel,
    out_type=[jax.ShapeDtypeStruct((NC, NP, DH), _f32),
              jax.ShapeDtypeStruct((NS, NP), _f32)],
    mesh=mesh,
    scratch_types=[
        pltpu.VMEM((NP,), _f32),
        pltpu.VMEM((NP,), _f32),
        pltpu.VMEM((NP,), _f32),
        pltpu.VMEM((EPW,), jnp.int32),
        [pltpu.VMEM((K,), jnp.int32)] * 3,
        [pltpu.VMEM((K,), jnp.int32)] * 3,
        [pltpu.VMEM((K, DH), _f32)] * 3,
        pltpu.VMEM_SHARED((NP, DH), _f32),
        [pltpu.SemaphoreType.DMA] * 3,
        [pltpu.SemaphoreType.DMA] * 3,
    ],
    compiler_params=pltpu.CompilerParams(needs_layout_passes=False,
                                         use_tc_tiling_on_sc=False),
  )
  def _edge_phase(h_hbm, s_hbm, d_hbm, comb_hbm, num_out, den_out,
                  s_v, d_v, den_v, comb_v, src_v, dst_v, rows_v, acc,
                  gsem, ssem):
      cc = lax.axis_index("c")
      ss = lax.axis_index("s")

      pltpu.sync_copy(s_hbm, s_v)
      pltpu.sync_copy(d_hbm, d_v)
      pltpu.sync_copy(comb_hbm.at[pl.ds(ss * EPW, EPW)], comb_v)

      zero16 = jnp.zeros((L,), _f32)

      def _zden(i, _):
          den_v[pl.ds(i * L, L)] = zero16
          return 0
      lax.fori_loop(0, NP // L, _zden, 0)

      def _zrows(i, _):
          for r in range(DH // L):
              rows_v[0][i, pl.ds(r * L, L)] = zero16
          return 0
      lax.fori_loop(0, K, _zrows, 0)
      for j in range(ROWS_PER_TILE // K):
          pltpu.sync_copy(rows_v[0],
                          acc.at[pl.ds(ss * ROWS_PER_TILE + j * K, K)])
      plsc.subcore_barrier()

      def _fire(bi, slot):
          base = bi * K
          for j in range(K // L):
              c = comb_v[pl.ds(base + j * L, L)]
              dst_v[slot][pl.ds(j * L, L)] = lax.shift_right_logical(c, 14)
              src_v[slot][pl.ds(j * L, L)] = lax.bitwise_and(c, 16383)
          pltpu.async_copy(h_hbm.at[cc].at[src_v[slot]], rows_v[slot],
                           gsem[slot])

      def _wait_gather(slot):
          pltpu.make_async_copy(h_hbm.at[cc].at[src_v[slot]], rows_v[slot],
                                gsem[slot]).wait()

      def _wait_scatter(slot):
          pltpu.make_async_copy(rows_v[slot], acc.at[dst_v[slot]],
                                ssem[slot]).wait()

      def _process(slot):
          for j in range(K // L):
              si = src_v[slot][pl.ds(j * L, L)]
              di = dst_v[slot][pl.ds(j * L, L)]
              e = plsc.load_gather(s_v, [si]) + plsc.load_gather(d_v, [di])
              e = jnp.where(e >= 0.0, e, 0.2 * e)
              p = jnp.exp(e)
              plsc.addupdate_scatter(den_v, [di], p)
              for l in range(L):
                  pv = p[l]
                  ei = j * L + l
                  for r in range(DH // L):
                      rows_v[slot][ei, pl.ds(r * L, L)] = (
                          rows_v[slot][ei, pl.ds(r * L, L)] * pv)

          pltpu.async_copy(rows_v[slot], acc.at[dst_v[slot]], ssem[slot],
                           add=True)

      _fire(0, 0)
      _fire(1, 1)

      def _tri(g, _):
          for t in range(3):
              i = 3 * g + t
              _wait_gather(t)
              _process(t)
              nxt = (t + 2) % 3
              if t == 0:
                  @pl.when(g > 0)
                  def _w():
                      _wait_scatter(nxt)
              else:
                  _wait_scatter(nxt)

              @pl.when(i + 2 < NB)
              def _f():
                  _fire(i + 2, nxt)
          return 0
      lax.fori_loop(0, NB // 3, _tri, 0)
      _wait_scatter(2)

      plsc.subcore_barrier()

      @pl.when(cc == 0)
      def _wden():
          pltpu.sync_copy(den_v, den_out.at[ss])
      for j in range(ROWS_PER_TILE // K):
          off = ss * ROWS_PER_TILE + j * K
          pltpu.sync_copy(acc.at[pl.ds(off, K)], num_out.at[cc, pl.ds(off, K)])

  return _edge_phase


def kernel(x, edge_index, batch, W1, a_src1, a_dst1, b1, W2, a_src2, a_dst2, b2):
    loop = jnp.arange(N, dtype=jnp.int32)
    pad = EPAD - ETOT
    src = jnp.concatenate([edge_index[0], loop,
                           jnp.zeros((pad,), jnp.int32)])
    dst = jnp.concatenate([edge_index[1], loop,
                           jnp.full((pad,), DUMMY, jnp.int32)])
    comb = src | (dst << 14)

    x_pad = jnp.pad(x, ((0, NP - N), (0, 0)))
    batch_col = jnp.pad(batch, (0, NP - N), constant_values=G).reshape(NP, 1)
    as1 = a_src1.reshape(D, 1)
    ad1 = a_dst1.reshape(D, 1)
    as2 = a_src2.reshape(D, 1)
    ad2 = a_dst2.reshape(D, 1)
    b1r = b1.reshape(1, D)
    b2r = b2.reshape(1, D)

    edge_phase = _build_edge_phase()
    h1, s1, d1 = _mm_attn(x_pad, W1, as1, ad1)
    num1, den1 = edge_phase(h1, s1.reshape(NP), d1.reshape(NP), comb)
    h2, s2, d2 = _fin_mm_attn(num1, den1, b1r, W2, as2, ad2)
    num2, den2 = edge_phase(h2, s2.reshape(NP), d2.reshape(NP), comb)
    return _pool(num2, den2, b2r, batch_col)
